# baseline (speedup 1.0000x reference)
import functools

import jax
import jax.numpy as jnp
from jax import lax
from jax.experimental import pallas as pl
from jax.experimental.pallas import tpu as pltpu
from jax.experimental.pallas import tpu_sc as plsc

DIM = 2048
NEXP = 16
STOK = 4096
CAP = int(1.25 * STOK / NEXP)
TOK_BLK = 1024
LANES = 16


def _router_body(x_ref, wt_ref, rnd_ref, mt_ref, n1_ref):
    logits = jnp.dot(x_ref[...], wt_ref[...], preferred_element_type=jnp.float32)
    cols = lax.broadcasted_iota(jnp.int32, logits.shape, 1)
    l1 = jnp.max(logits, axis=1, keepdims=True)
    e1 = jnp.min(jnp.where(logits == l1, cols, NEXP), axis=1, keepdims=True)
    m1 = cols == e1
    lm = jnp.where(m1, -jnp.inf, logits)
    l2 = jnp.max(lm, axis=1, keepdims=True)
    e2 = jnp.min(jnp.where(lm == l2, cols, NEXP), axis=1, keepdims=True)
    g = 1.0 / (1.0 + jnp.exp(l2 - l1))
    rc = rnd_ref[...] < 2.0 * g
    m2 = (cols == e2) & rc
    m = jnp.where(m1, g, jnp.where(m2, -g, 0.0))
    mt_ref[...] = m.T

    @pl.when(pl.program_id(0) == 0)
    def _():
        n1_ref[...] = jnp.zeros_like(n1_ref)

    n1_ref[...] += jnp.sum(jnp.where(m1, 1, 0), axis=0, keepdims=True)


_router = pl.pallas_call(
    _router_body,
    grid=(STOK // TOK_BLK,),
    in_specs=[
        pl.BlockSpec((TOK_BLK, DIM), lambda i: (i, 0)),
        pl.BlockSpec((DIM, NEXP), lambda i: (0, 0)),
        pl.BlockSpec((TOK_BLK, 1), lambda i: (i, 0)),
    ],
    out_specs=[
        pl.BlockSpec((NEXP, TOK_BLK), lambda i: (0, i)),
        pl.BlockSpec((1, NEXP), lambda i: (0, 0)),
    ],
    out_shape=[
        jax.ShapeDtypeStruct((NEXP, STOK), jnp.float32),
        jax.ShapeDtypeStruct((1, NEXP), jnp.int32),
    ],
)


@functools.cache
def _build_assign():
    mesh = plsc.VectorSubcoreMesh(core_axis_name="c", subcore_axis_name="s")
    return functools.partial(
        pl.kernel,
        mesh=mesh,
        compiler_params=pltpu.CompilerParams(needs_layout_passes=False),
        out_type=jax.ShapeDtypeStruct((NEXP, STOK), jnp.float32),
        scratch_types=[
            pltpu.VMEM((STOK,), jnp.float32),
            pltpu.VMEM((NEXP,), jnp.int32),
            pltpu.VMEM((STOK,), jnp.float32),
        ],
    )(_assign_body)


def _assign_body(mt_hbm, n1_hbm, out_hbm, row_v, n1_v, col_v):
    cid = lax.axis_index("c")
    sid = lax.axis_index("s")
    nchunks = STOK // LANES

    @pl.when(cid == 0)
    def _():
        e = sid
        pltpu.sync_copy(mt_hbm.at[e], row_v)
        pltpu.sync_copy(n1_hbm, n1_v)
        e_vec = jnp.zeros((LANES,), jnp.int32) + e
        n1_tot = plsc.load_gather(n1_v, [e_vec])

        def both(k, carry):
            c1, c2 = carry
            cv = row_v[pl.ds(k * LANES, LANES)]
            m1 = cv > 0.0
            m2 = cv < 0.0
            inc1 = jnp.where(m1, 1, 0)
            inc2 = jnp.where(m2, 1, 0)
            pc1 = jnp.cumsum(inc1)
            pc2 = jnp.cumsum(inc2)
            ok1 = m1 & ((pc1 - inc1 + c1) < CAP)
            ok2 = m2 & ((pc2 - inc2 + c2) < CAP)
            col_v[pl.ds(k * LANES, LANES)] = jnp.where(
                ok1, cv, jnp.where(ok2, -cv, 0.0)
            )
            return (
                c1 + plsc.all_reduce_population_count(m1),
                c2 + plsc.all_reduce_population_count(m2),
            )

        z = jnp.zeros((LANES,), jnp.int32)
        lax.fori_loop(0, nchunks, both, (z, n1_tot), unroll=4)
        pltpu.sync_copy(col_v, out_hbm.at[e])


def kernel(x, W_g):
    try:
        with jax.ensure_compile_time_eval():
            rnd = jax.random.uniform(jax.random.key(42), (x.shape[0],), dtype=jnp.float32).reshape(-1, 1)
    except Exception:
        rnd = jax.random.uniform(jax.random.key(42), (x.shape[0],), dtype=jnp.float32).reshape(-1, 1)
    mt, n1 = _router(x, W_g.T, rnd)
    combine_t = _build_assign()(mt, n1.reshape(NEXP))
    return combine_t.T
